# baseline (speedup 1.0000x reference)
import functools

import jax
import jax.numpy as jnp
from jax import lax
from jax.experimental import pallas as pl
from jax.experimental.pallas import tpu as pltpu
from jax.experimental.pallas import tpu_sc as plsc

_LANES = 16
_SCALE = 8.0
_NBUF = 4


def _emb_call(V, D, NW, CH, n_units):
    mesh = plsc.VectorSubcoreMesh(core_axis_name="c", subcore_axis_name="s")
    num_cores = mesh.num_cores
    K = n_units // _NBUF
    DT = D // 8

    @functools.partial(
        pl.kernel,
        out_type=jax.ShapeDtypeStruct((n_units, DT, NW, 8, CH), jnp.float32),
        mesh=mesh,
        scratch_types=[
            pltpu.VMEM((n_units // 8, 8, CH), jnp.int32),
            [pltpu.VMEM((CH, D), jnp.float32) for _ in range(_NBUF)],
            [pltpu.VMEM((DT, 8, CH), jnp.float32) for _ in range(_NBUF)],
            [pltpu.SemaphoreType.DMA for _ in range(_NBUF)],
            [pltpu.SemaphoreType.DMA for _ in range(_NBUF)],
        ],
        compiler_params=pltpu.CompilerParams(
            use_tc_tiling_on_sc=False, needs_layout_passes=False
        ),
    )
    def emb_kernel(idx_hbm, table_hbm, out_hbm, idx_v, rows_g, tbuf_s, gsem, ssem):
        wid = lax.axis_index("s") * num_cores + lax.axis_index("c")
        pltpu.sync_copy(idx_hbm.at[:, wid], idx_v)

        ids = [
            lax.iota(jnp.int32, _LANES) + g * _LANES for g in range(CH // _LANES)
        ]

        def idx_row(u):
            return idx_v.at[u // 8, u % 8]

        def transpose_scale(b):
            @pl.loop(0, DT)
            def _dt(dt):
                base_d = dt * 8
                for dr in range(8):
                    dcol = jnp.full((_LANES,), base_d + dr, jnp.int32)
                    for g in range(CH // _LANES):
                        v = plsc.load_gather(rows_g[b], [ids[g], dcol])
                        tbuf_s[b][dt, dr, pl.ds(g * _LANES, _LANES)] = v * _SCALE

        for b in range(_NBUF):
            pltpu.async_copy(table_hbm.at[idx_row(b)], rows_g[b], gsem[b])

        @pl.loop(0, K)
        def _block(k):
            for b in range(_NBUF):
                u = k * _NBUF + b
                pltpu.make_async_copy(
                    table_hbm.at[idx_row(u)], rows_g[b], gsem[b]
                ).wait()

                @pl.when(k > 0)
                def _wait_store():
                    pltpu.make_async_copy(
                        tbuf_s[b], out_hbm.at[u - _NBUF, :, wid], ssem[b]
                    ).wait()

                transpose_scale(b)
                pltpu.async_copy(tbuf_s[b], out_hbm.at[u, :, wid], ssem[b])

                @pl.when(k < K - 1)
                def _next_gather():
                    pltpu.async_copy(
                        table_hbm.at[idx_row(u + _NBUF)], rows_g[b], gsem[b]
                    )

        for b in range(_NBUF):
            pltpu.make_async_copy(
                tbuf_s[b],
                out_hbm.at[(K - 1) * _NBUF + b, :, wid],
                ssem[b],
            ).wait()

    return emb_kernel


def kernel(inputs, table):
    B0, B1 = inputs.shape
    V, D = table.shape
    NW = 32
    CH = 128

    idx_phys = (
        inputs.astype(jnp.int32)
        .reshape(NW, CH, B1 // 8, 8)
        .transpose(2, 0, 3, 1)
    )
    out5 = _emb_call(V, D, NW, CH, B1)(idx_phys, table)
    return out5.transpose(2, 4, 0, 1, 3).reshape(B0, B1, D)


# speedup vs baseline: 1.7645x; 1.7645x over previous
import functools

import jax
import jax.numpy as jnp
from jax import lax
from jax.experimental import pallas as pl
from jax.experimental.pallas import tpu as pltpu
from jax.experimental.pallas import tpu_sc as plsc

_LANES = 16
_SCALE = 8.0
_NBUF = 4


def _emb_call(V, D, NW, CH, n_units):
    mesh = plsc.VectorSubcoreMesh(core_axis_name="c", subcore_axis_name="s")
    num_cores = mesh.num_cores
    K = n_units // _NBUF
    DT = D // 8

    @functools.partial(
        pl.kernel,
        out_type=jax.ShapeDtypeStruct((n_units, DT, NW, 8, CH), jnp.float32),
        mesh=mesh,
        scratch_types=[
            pltpu.VMEM((n_units // 8, 8, CH), jnp.int32),
            [pltpu.VMEM((CH, D), jnp.float32) for _ in range(_NBUF)],
            [pltpu.VMEM((DT, 8, CH + 1), jnp.float32) for _ in range(_NBUF)],
            [pltpu.SemaphoreType.DMA for _ in range(_NBUF)],
            [pltpu.SemaphoreType.DMA for _ in range(_NBUF)],
        ],
        compiler_params=pltpu.CompilerParams(
            use_tc_tiling_on_sc=False, needs_layout_passes=False
        ),
    )
    def emb_kernel(idx_hbm, table_hbm, out_hbm, idx_v, rows_g, tbuf_s, gsem, ssem):
        wid = lax.axis_index("s") * num_cores + lax.axis_index("c")
        pltpu.sync_copy(idx_hbm.at[:, wid], idx_v)

        iota = lax.iota(jnp.int32, _LANES)
        dt_vecs = [(iota + q * _LANES) // 8 for q in range(D // _LANES)]
        dr_vecs = [(iota + q * _LANES) % 8 for q in range(D // _LANES)]

        def idx_row(u):
            return idx_v.at[u // 8, u % 8]

        def transpose_scale(b):
            @pl.loop(0, CH, unroll=2)
            def _r(r):
                bcol = jnp.full((_LANES,), r, jnp.int32)
                for q in range(D // _LANES):
                    v = rows_g[b][r, pl.ds(q * _LANES, _LANES)]
                    plsc.store_scatter(
                        tbuf_s[b], [dt_vecs[q], dr_vecs[q], bcol], v * _SCALE
                    )

        for b in range(_NBUF):
            pltpu.async_copy(table_hbm.at[idx_row(b)], rows_g[b], gsem[b])

        @pl.loop(0, K)
        def _block(k):
            for b in range(_NBUF):
                u = k * _NBUF + b
                pltpu.make_async_copy(
                    table_hbm.at[idx_row(u)], rows_g[b], gsem[b]
                ).wait()

                @pl.when(k > 0)
                def _wait_store():
                    pltpu.make_async_copy(
                        tbuf_s[b].at[:, :, pl.ds(0, CH)],
                        out_hbm.at[u - _NBUF, :, wid],
                        ssem[b],
                    ).wait()

                transpose_scale(b)
                pltpu.async_copy(
                    tbuf_s[b].at[:, :, pl.ds(0, CH)],
                    out_hbm.at[u, :, wid],
                    ssem[b],
                )

                @pl.when(k < K - 1)
                def _next_gather():
                    pltpu.async_copy(
                        table_hbm.at[idx_row(u + _NBUF)], rows_g[b], gsem[b]
                    )

        for b in range(_NBUF):
            pltpu.make_async_copy(
                tbuf_s[b].at[:, :, pl.ds(0, CH)],
                out_hbm.at[(K - 1) * _NBUF + b, :, wid],
                ssem[b],
            ).wait()

    return emb_kernel


def kernel(inputs, table):
    B0, B1 = inputs.shape
    V, D = table.shape
    NW = 32
    CH = 128

    idx_phys = (
        inputs.astype(jnp.int32)
        .reshape(NW, CH, B1 // 8, 8)
        .transpose(2, 0, 3, 1)
    )
    out5 = _emb_call(V, D, NW, CH, B1)(idx_phys, table)
    return out5.transpose(2, 4, 0, 1, 3).reshape(B0, B1, D)


# speedup vs baseline: 1.7683x; 1.0022x over previous
import functools

import jax
import jax.numpy as jnp
from jax import lax
from jax.experimental import pallas as pl
from jax.experimental.pallas import tpu as pltpu
from jax.experimental.pallas import tpu_sc as plsc

_LANES = 16
_SCALE = 8.0
_NBUF = 5


def _emb_call(V, D, NW, CH, n_units):
    mesh = plsc.VectorSubcoreMesh(core_axis_name="c", subcore_axis_name="s")
    num_cores = mesh.num_cores
    K = n_units // _NBUF
    DT = D // 8

    @functools.partial(
        pl.kernel,
        out_type=jax.ShapeDtypeStruct((n_units, DT, NW, 8, CH), jnp.float32),
        mesh=mesh,
        scratch_types=[
            pltpu.VMEM((n_units // 8, 8, CH), jnp.int32),
            [pltpu.VMEM((CH, D), jnp.float32) for _ in range(_NBUF)],
            [pltpu.VMEM((DT, 8, CH + 1), jnp.float32) for _ in range(_NBUF)],
            [pltpu.SemaphoreType.DMA for _ in range(_NBUF)],
            [pltpu.SemaphoreType.DMA for _ in range(_NBUF)],
        ],
        compiler_params=pltpu.CompilerParams(
            use_tc_tiling_on_sc=False, needs_layout_passes=False
        ),
    )
    def emb_kernel(idx_hbm, table_hbm, out_hbm, idx_v, rows_g, tbuf_s, gsem, ssem):
        wid = lax.axis_index("s") * num_cores + lax.axis_index("c")
        pltpu.sync_copy(idx_hbm.at[:, wid], idx_v)

        iota = lax.iota(jnp.int32, _LANES)
        dt_vecs = [(iota + q * _LANES) // 8 for q in range(D // _LANES)]
        dr_vecs = [(iota + q * _LANES) % 8 for q in range(D // _LANES)]

        def idx_row(u):
            return idx_v.at[u // 8, u % 8]

        def transpose_scale(b):
            @pl.loop(0, CH, unroll=4)
            def _r(r):
                bcol = jnp.full((_LANES,), r, jnp.int32)
                for q in range(D // _LANES):
                    v = rows_g[b][r, pl.ds(q * _LANES, _LANES)]
                    plsc.store_scatter(
                        tbuf_s[b], [dt_vecs[q], dr_vecs[q], bcol], v * _SCALE
                    )

        for b in range(_NBUF):
            pltpu.async_copy(table_hbm.at[idx_row(b)], rows_g[b], gsem[b])

        @pl.loop(0, K)
        def _block(k):
            for b in range(_NBUF):
                u = k * _NBUF + b
                pltpu.make_async_copy(
                    table_hbm.at[idx_row(u)], rows_g[b], gsem[b]
                ).wait()

                @pl.when(k > 0)
                def _wait_store():
                    pltpu.make_async_copy(
                        tbuf_s[b].at[:, :, pl.ds(0, CH)],
                        out_hbm.at[u - _NBUF, :, wid],
                        ssem[b],
                    ).wait()

                transpose_scale(b)
                pltpu.async_copy(
                    tbuf_s[b].at[:, :, pl.ds(0, CH)],
                    out_hbm.at[u, :, wid],
                    ssem[b],
                )

                @pl.when(k < K - 1)
                def _next_gather():
                    pltpu.async_copy(
                        table_hbm.at[idx_row(u + _NBUF)], rows_g[b], gsem[b]
                    )

        for b in range(_NBUF):
            pltpu.make_async_copy(
                tbuf_s[b].at[:, :, pl.ds(0, CH)],
                out_hbm.at[(K - 1) * _NBUF + b, :, wid],
                ssem[b],
            ).wait()

    return emb_kernel


def kernel(inputs, table):
    B0, B1 = inputs.shape
    V, D = table.shape
    NW = 32
    CH = 128

    idx_phys = (
        inputs.astype(jnp.int32)
        .reshape(NW, CH, B1 // 8, 8)
        .transpose(2, 0, 3, 1)
    )
    out5 = _emb_call(V, D, NW, CH, B1)(idx_phys, table)
    return out5.transpose(2, 4, 0, 1, 3).reshape(B0, B1, D)


# speedup vs baseline: 1.7721x; 1.0021x over previous
import functools

import jax
import jax.numpy as jnp
from jax import lax
from jax.experimental import pallas as pl
from jax.experimental.pallas import tpu as pltpu
from jax.experimental.pallas import tpu_sc as plsc

_LANES = 16
_SCALE = 8.0
_NBUF = 5


def _emb_call(V, D, NW, CH, n_units):
    mesh = plsc.VectorSubcoreMesh(core_axis_name="c", subcore_axis_name="s")
    num_cores = mesh.num_cores
    K = n_units // _NBUF
    DT = D // 8

    @functools.partial(
        pl.kernel,
        out_type=jax.ShapeDtypeStruct((n_units, DT, NW, 8, CH), jnp.float32),
        mesh=mesh,
        scratch_types=[
            pltpu.VMEM((n_units // 8, 8, CH), jnp.int32),
            [pltpu.VMEM((CH, D), jnp.float32) for _ in range(_NBUF)],
            [pltpu.VMEM((DT, 8, CH + 1), jnp.float32) for _ in range(_NBUF)],
            [pltpu.SemaphoreType.DMA for _ in range(_NBUF)],
            [pltpu.SemaphoreType.DMA for _ in range(_NBUF)],
        ],
        compiler_params=pltpu.CompilerParams(
            use_tc_tiling_on_sc=False,
            needs_layout_passes=False,
            skip_device_barrier=True,
            disable_bounds_checks=True,
            disable_semaphore_checks=True,
        ),
    )
    def emb_kernel(idx_hbm, table_hbm, out_hbm, idx_v, rows_g, tbuf_s, gsem, ssem):
        wid = lax.axis_index("s") * num_cores + lax.axis_index("c")
        pltpu.sync_copy(idx_hbm.at[:, wid], idx_v)

        iota = lax.iota(jnp.int32, _LANES)
        dt_vecs = [(iota + q * _LANES) // 8 for q in range(D // _LANES)]
        dr_vecs = [(iota + q * _LANES) % 8 for q in range(D // _LANES)]

        def idx_row(u):
            return idx_v.at[u // 8, u % 8]

        def transpose_scale(b):
            @pl.loop(0, CH, unroll=4)
            def _r(r):
                bcol = jnp.full((_LANES,), r, jnp.int32)
                for q in range(D // _LANES):
                    v = rows_g[b][r, pl.ds(q * _LANES, _LANES)]
                    plsc.store_scatter(
                        tbuf_s[b], [dt_vecs[q], dr_vecs[q], bcol], v * _SCALE
                    )

        for b in range(_NBUF):
            pltpu.async_copy(table_hbm.at[idx_row(b)], rows_g[b], gsem[b])

        @pl.loop(0, K)
        def _block(k):
            for b in range(_NBUF):
                u = k * _NBUF + b
                pltpu.make_async_copy(
                    table_hbm.at[idx_row(u)], rows_g[b], gsem[b]
                ).wait()

                @pl.when(k > 0)
                def _wait_store():
                    pltpu.make_async_copy(
                        tbuf_s[b].at[:, :, pl.ds(0, CH)],
                        out_hbm.at[u - _NBUF, :, wid],
                        ssem[b],
                    ).wait()

                transpose_scale(b)
                pltpu.async_copy(
                    tbuf_s[b].at[:, :, pl.ds(0, CH)],
                    out_hbm.at[u, :, wid],
                    ssem[b],
                )

                @pl.when(k < K - 1)
                def _next_gather():
                    pltpu.async_copy(
                        table_hbm.at[idx_row(u + _NBUF)], rows_g[b], gsem[b]
                    )

        for b in range(_NBUF):
            pltpu.make_async_copy(
                tbuf_s[b].at[:, :, pl.ds(0, CH)],
                out_hbm.at[(K - 1) * _NBUF + b, :, wid],
                ssem[b],
            ).wait()

    return emb_kernel


def kernel(inputs, table):
    B0, B1 = inputs.shape
    V, D = table.shape
    NW = 32
    CH = 128

    idx_phys = (
        inputs.astype(jnp.int32)
        .reshape(NW, CH, B1 // 8, 8)
        .transpose(2, 0, 3, 1)
    )
    out5 = _emb_call(V, D, NW, CH, B1)(idx_phys, table)
    return out5.transpose(2, 4, 0, 1, 3).reshape(B0, B1, D)
